# Initial kernel scaffold; baseline (speedup 1.0000x reference)
import jax
import jax.numpy as jnp
from jax.experimental import pallas as pl


def kernel(token_ids, embed_weight):
    raise NotImplementedError("write your pallas kernel here")


# speedup vs baseline: 7.3872x; 7.3872x over previous
import jax
import jax.numpy as jnp
from jax.experimental import pallas as pl
from jax.experimental.pallas import tpu as pltpu
from jax.experimental.pallas import tpu_sc as plsc


def kernel(token_ids, embed_weight):
    batch, seq = token_ids.shape
    vocab, embed_dim = embed_weight.shape
    num_indices = batch * seq

    window = 128
    assert num_indices % window == 0
    grid = num_indices // window

    indices = token_ids.reshape(1, num_indices).astype(jnp.int32)

    mesh = plsc.VectorSubcoreMesh(
        core_axis_name="core", subcore_axis_name="subcore"
    )

    @pl.kernel(
        out_type=jax.ShapeDtypeStruct((num_indices, embed_dim),
                                      embed_weight.dtype),
        mesh=mesh,
    )
    def sc_gather(table_hbm, idx_hbm, out_hbm):
        def body(i_vmem, o_vmem):
            pltpu.sync_copy(table_hbm.at[i_vmem.at[0]], o_vmem)

        pltpu.emit_pipeline(
            body,
            grid=(grid,),
            in_specs=[pl.BlockSpec((1, window), index_map=lambda i: (0, i))],
            out_specs=[pl.BlockSpec((window, embed_dim),
                                    index_map=lambda i: (i, 0))],
            core_axis_name=("core", "subcore"),
            dimension_semantics=(pltpu.PARALLEL,),
        )(idx_hbm, out_hbm)

    out = sc_gather(embed_weight, indices)
    return out.reshape(batch, seq, embed_dim)


# speedup vs baseline: 9.1464x; 1.2382x over previous
import jax
import jax.numpy as jnp
from jax.experimental import pallas as pl
from jax.experimental.pallas import tpu as pltpu
from jax.experimental.pallas import tpu_sc as plsc


def kernel(token_ids, embed_weight):
    batch, seq = token_ids.shape
    vocab, embed_dim = embed_weight.shape
    num_indices = batch * seq

    window = 256
    assert num_indices % window == 0
    grid = num_indices // window

    indices = token_ids.reshape(1, num_indices).astype(jnp.int32)

    mesh = plsc.VectorSubcoreMesh(
        core_axis_name="core", subcore_axis_name="subcore"
    )

    @pl.kernel(
        out_type=jax.ShapeDtypeStruct((num_indices, embed_dim),
                                      embed_weight.dtype),
        mesh=mesh,
    )
    def sc_gather(table_hbm, idx_hbm, out_hbm):
        def body(i_vmem, o_vmem):
            pltpu.sync_copy(table_hbm.at[i_vmem.at[0]], o_vmem)

        pltpu.emit_pipeline(
            body,
            grid=(grid,),
            in_specs=[pl.BlockSpec((1, window), index_map=lambda i: (0, i))],
            out_specs=[pl.BlockSpec((window, embed_dim),
                                    index_map=lambda i: (i, 0))],
            core_axis_name=("core", "subcore"),
            dimension_semantics=(pltpu.PARALLEL,),
        )(idx_hbm, out_hbm)

    out = sc_gather(embed_weight, indices)
    return out.reshape(batch, seq, embed_dim)


# speedup vs baseline: 9.1815x; 1.0038x over previous
import jax
import jax.numpy as jnp
from jax import lax
from jax.experimental import pallas as pl
from jax.experimental.pallas import tpu as pltpu
from jax.experimental.pallas import tpu_sc as plsc

_NUM_CORES = 2
_NUM_SUBCORES = 16
_NUM_WORKERS = _NUM_CORES * _NUM_SUBCORES


def kernel(token_ids, embed_weight):
    batch, seq = token_ids.shape
    vocab, embed_dim = embed_weight.shape
    num_indices = batch * seq

    b_per_w = num_indices // _NUM_WORKERS
    window = 128
    nbuf = 4
    steps = b_per_w // window
    assert b_per_w % window == 0 and steps % nbuf == 0 and steps >= 2 * nbuf

    indices = token_ids.reshape(num_indices).astype(jnp.int32)

    mesh = plsc.VectorSubcoreMesh(
        core_axis_name="core", subcore_axis_name="subcore"
    )

    @pl.kernel(
        out_type=jax.ShapeDtypeStruct((num_indices, embed_dim),
                                      embed_weight.dtype),
        mesh=mesh,
        scratch_types=[
            pltpu.VMEM((b_per_w,), jnp.int32),
            pltpu.VMEM((nbuf, window, embed_dim), jnp.float32),
            pltpu.SemaphoreType.DMA((nbuf,)),
            pltpu.SemaphoreType.DMA((nbuf,)),
        ],
    )
    def sc_gather(table_hbm, idx_hbm, out_hbm, idx_v, rows_v, gsem, wsem):
        wid = lax.axis_index("core") * _NUM_SUBCORES + lax.axis_index(
            "subcore")
        base = wid * b_per_w
        pltpu.sync_copy(idx_hbm.at[pl.ds(base, b_per_w)], idx_v)

        def start_gather(s, b):
            pltpu.async_copy(
                table_hbm.at[idx_v.at[pl.ds(s * window, window)]],
                rows_v.at[b], gsem.at[b])

        def wait_gather(b):
            pltpu.make_async_copy(
                table_hbm.at[pl.ds(0, window)], rows_v.at[b],
                gsem.at[b]).wait()

        def start_wb(s, b):
            pltpu.async_copy(
                rows_v.at[b],
                out_hbm.at[pl.ds(base + s * window, window)], wsem.at[b])

        def wait_wb(b):
            pltpu.make_async_copy(
                rows_v.at[b], out_hbm.at[pl.ds(0, window)],
                wsem.at[b]).wait()

        for s in range(nbuf):
            start_gather(s, s)
            if s >= 2:
                b2 = s - 2
                wait_gather(b2)
                start_wb(b2, b2)

        @pl.loop(nbuf, steps, step=nbuf)
        def _(g):
            for b in range(nbuf):
                s = g + b
                wait_wb(b)
                start_gather(s, b)
                b2 = (b - 2) % nbuf
                wait_gather(b2)
                start_wb(s - 2, b2)

        for off in (2, 1):
            b2 = (steps - off) % nbuf
            wait_gather(b2)
            start_wb(steps - off, b2)
        for b in range(nbuf):
            wait_wb(b)

    out = sc_gather(embed_weight, indices)
    return out.reshape(batch, seq, embed_dim)


# speedup vs baseline: 9.6158x; 1.0473x over previous
import jax
import jax.numpy as jnp
from jax import lax
from jax.experimental import pallas as pl
from jax.experimental.pallas import tpu as pltpu
from jax.experimental.pallas import tpu_sc as plsc

_NUM_CORES = 2
_NUM_SUBCORES = 16
_NUM_WORKERS = _NUM_CORES * _NUM_SUBCORES


def kernel(token_ids, embed_weight):
    batch, seq = token_ids.shape
    vocab, embed_dim = embed_weight.shape
    num_indices = batch * seq

    b_per_w = num_indices // _NUM_WORKERS
    window = 128
    nbuf = 4
    nslot = 2
    steps = b_per_w // window
    assert b_per_w % window == 0 and steps % nbuf == 0 and steps >= 2 * nbuf

    indices = token_ids.reshape(num_indices).astype(jnp.int32)

    mesh = plsc.VectorSubcoreMesh(
        core_axis_name="core", subcore_axis_name="subcore"
    )

    @pl.kernel(
        out_type=jax.ShapeDtypeStruct((num_indices, embed_dim),
                                      embed_weight.dtype),
        mesh=mesh,
        scratch_types=[
            pltpu.VMEM((b_per_w,), jnp.int32),
            pltpu.VMEM((nbuf, window, embed_dim), jnp.float32),
            pltpu.VMEM_SHARED((_NUM_SUBCORES, nslot, window, embed_dim),
                              jnp.float32),
            pltpu.SemaphoreType.DMA((nbuf,)),
            pltpu.SemaphoreType.DMA((nslot,)),
            pltpu.SemaphoreType.DMA((nslot,)),
        ],
    )
    def sc_gather(table_hbm, idx_hbm, out_hbm, idx_v, rows_v, stage_v,
                  gsem, csem, hsem):
        wid = lax.axis_index("core") * _NUM_SUBCORES + lax.axis_index(
            "subcore")
        sid = lax.axis_index("subcore")
        base = wid * b_per_w
        pltpu.sync_copy(idx_hbm.at[pl.ds(base, b_per_w)], idx_v)

        def start_g(s, b):
            pltpu.async_copy(
                table_hbm.at[idx_v.at[pl.ds(s * window, window)]],
                rows_v.at[b], gsem.at[b])

        def wait_g(b):
            pltpu.make_async_copy(
                table_hbm.at[pl.ds(0, window)], rows_v.at[b],
                gsem.at[b]).wait()

        def start_c(b, p):
            pltpu.async_copy(rows_v.at[b], stage_v.at[sid, p], csem.at[p])

        def wait_c(p):
            pltpu.make_async_copy(rows_v.at[0], stage_v.at[sid, p],
                                  csem.at[p]).wait()

        def start_h(s, p):
            pltpu.async_copy(
                stage_v.at[sid, p],
                out_hbm.at[pl.ds(base + s * window, window)], hsem.at[p])

        def wait_h(p):
            pltpu.make_async_copy(
                stage_v.at[sid, p], out_hbm.at[pl.ds(0, window)],
                hsem.at[p]).wait()

        start_g(0, 0)
        start_g(1, 1)
        start_g(2, 2)
        wait_g(0)
        start_c(0, 0)
        start_g(3, 3)
        wait_g(1)
        start_c(1, 1)
        wait_c(0)
        start_h(0, 0)

        @pl.loop(4, steps, step=nbuf)
        def _(g):
            for b in range(nbuf):
                s = g + b
                b2 = (b - 2) % nbuf
                p = b % nslot
                q = (b + 1) % nslot
                start_g(s, b)
                wait_g(b2)
                wait_h(p)
                start_c(b2, p)
                wait_c(q)
                start_h(s - 3, q)

        wait_g((steps - 2) % nbuf)
        wait_h(0)
        start_c((steps - 2) % nbuf, 0)
        wait_c(1)
        start_h(steps - 3, 1)

        wait_g((steps - 1) % nbuf)
        wait_h(1)
        start_c((steps - 1) % nbuf, 1)
        wait_c(0)
        start_h(steps - 2, 0)

        wait_c(1)
        start_h(steps - 1, 1)

        wait_h(0)
        wait_h(1)

    out = sc_gather(embed_weight, indices)
    return out.reshape(batch, seq, embed_dim)
